# baseline (speedup 1.0000x reference)
import functools

import jax
import jax.numpy as jnp
from jax import lax
from jax.experimental import pallas as pl
from jax.experimental.pallas import tpu as pltpu
from jax.experimental.pallas import tpu_sc as plsc

_BETA = 0.001
_NSPLITS = 8
_NUM_EMB = 1024
_EMB_DIM = 32

_BT = 1024


def _tc_body(z_ref, cb_ref, idx_ref, loss_ref, cn_ref):
    b = pl.program_id(0)

    @pl.when(b == 0)
    def _():
        cn_ref[...] = jnp.sum(cb_ref[...] * cb_ref[...], axis=2)

    cols = []
    lsum = jnp.float32(0.0)
    iotaf = lax.broadcasted_iota(
        jnp.int32, (z_ref.shape[0], _NUM_EMB), 1).astype(jnp.float32)
    for i in range(_NSPLITS):
        zc = z_ref[:, i * _EMB_DIM:(i + 1) * _EMB_DIM]
        w = cb_ref[i]
        s2 = lax.dot_general(zc * (-2.0), w, (((1,), (1,)), ((), ())),
                             preferred_element_type=jnp.float32)
        rn = jnp.sum(zc * zc, axis=1, keepdims=True)
        cn = cn_ref[i][None, :]
        d = (rn + cn) + s2
        m = jnp.min(d, axis=1, keepdims=True)
        idxf = jnp.min(jnp.where(d == m, iotaf, float(_NUM_EMB)), axis=1)
        idx = idxf.astype(jnp.int32)
        cols.append((idx + i * _NUM_EMB)[:, None])
        lsum = lsum + jnp.sum(m)
    idx_ref[...] = jnp.concatenate(cols, axis=1)
    prev = jnp.where(b == 0, 0.0, loss_ref[0, 0])
    loss_ref[0, 0] = prev + lsum


def _tc_indices(z, codebooks, nh, off):
    grid = nh // _BT
    return pl.pallas_call(
        _tc_body,
        grid=(grid,),
        in_specs=[
            pl.BlockSpec((_BT, _NSPLITS * _EMB_DIM), lambda b: (b + off, 0)),
            pl.BlockSpec((_NSPLITS, _NUM_EMB, _EMB_DIM), lambda b: (0, 0, 0)),
        ],
        out_specs=[
            pl.BlockSpec((_BT, _NSPLITS), lambda b: (b, 0)),
            pl.BlockSpec(memory_space=pltpu.SMEM, block_shape=(1, 1),
                         index_map=lambda b: (0, 0)),
        ],
        out_shape=[
            jax.ShapeDtypeStruct((nh, _NSPLITS), jnp.int32),
            jax.ShapeDtypeStruct((1, 1), jnp.float32),
        ],
        scratch_shapes=[pltpu.VMEM((_NSPLITS, _NUM_EMB), jnp.float32)],
    )(z, codebooks)


_SC_CHUNK = 1024
_SC_IDXC = 128


def _make_sc_gather(b_total):
    info = plsc.get_sparse_core_info()
    nw = info.num_cores * info.num_subcores
    b_per_w = b_total // nw
    nch = b_per_w // _SC_CHUNK
    mesh = plsc.VectorSubcoreMesh(core_axis_name="c", subcore_axis_name="s")

    @functools.partial(
        pl.kernel, mesh=mesh,
        out_type=jax.ShapeDtypeStruct((b_total, _EMB_DIM), jnp.float32),
        scratch_types=[
            pltpu.VMEM((_SC_CHUNK,), jnp.int32),
            pltpu.VMEM((_SC_CHUNK, _EMB_DIM), jnp.float32),
            pltpu.SemaphoreType.DMA,
        ],
        compiler_params=pltpu.CompilerParams(use_tc_tiling_on_sc=False),
    )
    def _sc_gather(table_hbm, idx_hbm, out_hbm, idx_v, rows_v, sem):
        wid = lax.axis_index("s") * info.num_cores + lax.axis_index("c")
        base = wid * b_per_w

        def chunk(jc, carry):
            row0 = base + jc * _SC_CHUNK
            pltpu.sync_copy(idx_hbm.at[pl.ds(row0, _SC_CHUNK)], idx_v)
            copies = []
            for k in range(_SC_CHUNK // _SC_IDXC):
                copies.append(pltpu.async_copy(
                    table_hbm.at[idx_v.at[pl.ds(k * _SC_IDXC, _SC_IDXC)]],
                    rows_v.at[pl.ds(k * _SC_IDXC, _SC_IDXC)],
                    sem))
            for c in copies:
                c.wait()
            pltpu.sync_copy(rows_v, out_hbm.at[pl.ds(row0, _SC_CHUNK)])
            return carry

        lax.fori_loop(0, nch, chunk, 0)

    return _sc_gather


def kernel(z, codebooks):
    n = z.shape[0]
    nh = n // 2
    table = codebooks.reshape(_NSPLITS * _NUM_EMB, _EMB_DIM)
    gather = _make_sc_gather(nh * _NSPLITS)
    idx1, l1 = _tc_indices(z, codebooks, nh, 0)
    rows1 = gather(table, idx1.reshape(-1))
    idx2, l2 = _tc_indices(z, codebooks, nh, nh // _BT)
    rows2 = gather(table, idx2.reshape(-1))
    zq = jnp.concatenate(
        [rows1.reshape(nh, _NSPLITS * _EMB_DIM),
         rows2.reshape(nh, _NSPLITS * _EMB_DIM)], axis=0)
    loss = (1.0 + _BETA) * (l1[0, 0] + l2[0, 0]) / jnp.float32(n * _EMB_DIM)
    return zq, loss


# speedup vs baseline: 1.0194x; 1.0194x over previous
import functools

import jax
import jax.numpy as jnp
from jax import lax
from jax.experimental import pallas as pl
from jax.experimental.pallas import tpu as pltpu
from jax.experimental.pallas import tpu_sc as plsc

_BETA = 0.001
_NSPLITS = 8
_NUM_EMB = 1024
_EMB_DIM = 32

_BT = 1024


def _tc_body(z_ref, cb_ref, idx_ref, loss_ref, cn_ref):
    b = pl.program_id(0)

    @pl.when(b == 0)
    def _():
        cn_ref[...] = jnp.sum(cb_ref[...] * cb_ref[...], axis=2)

    cols = []
    lsum = jnp.float32(0.0)
    iotaf = lax.broadcasted_iota(
        jnp.int32, (z_ref.shape[0], _NUM_EMB), 1).astype(jnp.float32)
    for i in range(_NSPLITS):
        zc = z_ref[:, i * _EMB_DIM:(i + 1) * _EMB_DIM]
        w = cb_ref[i]
        s2 = lax.dot_general(zc * (-2.0), w, (((1,), (1,)), ((), ())),
                             preferred_element_type=jnp.float32)
        rn = jnp.sum(zc * zc, axis=1, keepdims=True)
        cn = cn_ref[i][None, :]
        d = (rn + cn) + s2
        m = jnp.min(d, axis=1, keepdims=True)
        idxf = jnp.min(jnp.where(d == m, iotaf, float(_NUM_EMB)), axis=1)
        idx = idxf.astype(jnp.int32)
        cols.append((idx + i * _NUM_EMB)[:, None])
        lsum = lsum + jnp.sum(m)
    idx_ref[...] = jnp.concatenate(cols, axis=1)
    prev = jnp.where(b == 0, 0.0, loss_ref[0, 0])
    loss_ref[0, 0] = prev + lsum


def _tc_indices(z, codebooks, nh, off):
    grid = nh // _BT
    return pl.pallas_call(
        _tc_body,
        grid=(grid,),
        in_specs=[
            pl.BlockSpec((_BT, _NSPLITS * _EMB_DIM), lambda b: (b + off, 0)),
            pl.BlockSpec((_NSPLITS, _NUM_EMB, _EMB_DIM), lambda b: (0, 0, 0)),
        ],
        out_specs=[
            pl.BlockSpec((_BT, _NSPLITS), lambda b: (b, 0)),
            pl.BlockSpec(memory_space=pltpu.SMEM, block_shape=(1, 1),
                         index_map=lambda b: (0, 0)),
        ],
        out_shape=[
            jax.ShapeDtypeStruct((nh, _NSPLITS), jnp.int32),
            jax.ShapeDtypeStruct((1, 1), jnp.float32),
        ],
        scratch_shapes=[pltpu.VMEM((_NSPLITS, _NUM_EMB), jnp.float32)],
    )(z, codebooks)


_SC_CHUNK = 1024
_SC_IDXC = 128


def _make_sc_gather(b_total):
    info = plsc.get_sparse_core_info()
    nw = info.num_cores * info.num_subcores
    b_per_w = b_total // nw
    nch = b_per_w // _SC_CHUNK
    mesh = plsc.VectorSubcoreMesh(core_axis_name="c", subcore_axis_name="s")

    @functools.partial(
        pl.kernel, mesh=mesh,
        out_type=jax.ShapeDtypeStruct((b_total, _EMB_DIM), jnp.float32),
        scratch_types=[
            pltpu.VMEM((_SC_CHUNK,), jnp.int32),
            pltpu.VMEM((_SC_CHUNK, _EMB_DIM), jnp.float32),
            pltpu.SemaphoreType.DMA,
        ],
        compiler_params=pltpu.CompilerParams(use_tc_tiling_on_sc=False),
    )
    def _sc_gather(table_hbm, idx_hbm, out_hbm, idx_v, rows_v, sem):
        wid = lax.axis_index("s") * info.num_cores + lax.axis_index("c")
        base = wid * b_per_w

        def chunk(jc, carry):
            row0 = base + jc * _SC_CHUNK
            pltpu.sync_copy(idx_hbm.at[pl.ds(row0, _SC_CHUNK)], idx_v)
            copies = []
            for k in range(_SC_CHUNK // _SC_IDXC):
                copies.append(pltpu.async_copy(
                    table_hbm.at[idx_v.at[pl.ds(k * _SC_IDXC, _SC_IDXC)]],
                    rows_v.at[pl.ds(k * _SC_IDXC, _SC_IDXC)],
                    sem))
            for c in copies:
                c.wait()
            pltpu.sync_copy(rows_v, out_hbm.at[pl.ds(row0, _SC_CHUNK)])
            return carry

        lax.fori_loop(0, nch, chunk, 0)

    return _sc_gather


def kernel(z, codebooks):
    n = z.shape[0]
    idx, lsum = _tc_indices(z, codebooks, n, 0)
    table = codebooks.reshape(_NSPLITS * _NUM_EMB, _EMB_DIM)
    rows = _make_sc_gather(n * _NSPLITS)(table, idx.reshape(-1))
    zq = rows.reshape(n, _NSPLITS * _EMB_DIM)
    loss = (1.0 + _BETA) * lsum[0, 0] / jnp.float32(n * _EMB_DIM)
    return zq, loss
